# Initial kernel scaffold; baseline (speedup 1.0000x reference)
import functools
import jax
import jax.numpy as jnp
from jax import lax
from jax.experimental import pallas as pl
from jax.experimental.pallas import tpu as pltpu
from jax.experimental.pallas import tpu_sc as plsc

NC = 2
NS = 16
NW = NC * NS

TABLE_ROWS = 1_000_000
D = 32
B = 16384 * 50
IW = 128
NROWS = B // IW
ROWS_PER_W = NROWS // NW
K = 10
NCHUNK = ROWS_PER_W // K


def _gather_body(table_hbm, idx_hbm, out_hbm, idx_v, rows_v, sem):
    c = lax.axis_index("c")
    s = lax.axis_index("s")
    wid = s * NC + c
    base_row = wid * ROWS_PER_W

    def chunk(g, carry):
        r0 = base_row + g * K
        pltpu.sync_copy(idx_hbm.at[pl.ds(r0, K)], idx_v)
        copies = [
            pltpu.async_copy(
                table_hbm.at[idx_v.at[j]],
                rows_v.at[pl.ds(j * IW, IW)],
                sem,
            )
            for j in range(K)
        ]
        for cp in copies:
            cp.wait()
        pltpu.sync_copy(rows_v, out_hbm.at[pl.ds(r0 * IW, K * IW)])
        return carry

    lax.fori_loop(0, NCHUNK, chunk, 0)


@jax.jit
def _gather(data, idx2d):
    mesh = plsc.VectorSubcoreMesh(
        core_axis_name="c", subcore_axis_name="s",
        num_cores=NC, num_subcores=NS,
    )
    k = pl.kernel(
        _gather_body,
        out_type=jax.ShapeDtypeStruct((B, D), jnp.float32),
        mesh=mesh,
        scratch_types=[
            pltpu.VMEM((K, IW), jnp.int32),
            pltpu.VMEM((K * IW, D), jnp.float32),
            pltpu.SemaphoreType.DMA,
        ],
    )
    return k(data, idx2d)


def kernel(data, indices):
    idx2d = indices.astype(jnp.int32).reshape(NROWS, IW)
    out = _gather(data, idx2d)
    return out.reshape(indices.shape[0], indices.shape[1], D)


# speedup vs baseline: 1.0939x; 1.0939x over previous
import functools
import jax
import jax.numpy as jnp
from jax import lax
from jax.experimental import pallas as pl
from jax.experimental.pallas import tpu as pltpu
from jax.experimental.pallas import tpu_sc as plsc

NC = 2
NS = 16
NW = NC * NS

TABLE_ROWS = 1_000_000
D = 32
B = 16384 * 50
IW = 128
NROWS = B // IW
ROWS_PER_W = NROWS // NW
K = 8
NCHUNK = ROWS_PER_W // K


def _gather_body(table_hbm, idx_hbm, out_hbm, idx_v, rows_v, sem):
    c = lax.axis_index("c")
    s = lax.axis_index("s")
    wid = s * NC + c
    base_row = wid * ROWS_PER_W

    def chunk(g, carry):
        r0 = base_row + g * K
        pltpu.sync_copy(idx_hbm.at[pl.ds(r0, K)], idx_v)
        copies = [
            pltpu.async_copy(
                table_hbm.at[idx_v.at[j]],
                rows_v.at[pl.ds(j * IW, IW)],
                sem,
            )
            for j in range(K)
        ]
        for cp in copies:
            cp.wait()
        pltpu.sync_copy(rows_v, out_hbm.at[pl.ds(r0 * IW, K * IW)])
        return carry

    lax.fori_loop(0, NCHUNK, chunk, 0)


@jax.jit
def _gather(data, idx2d):
    mesh = plsc.VectorSubcoreMesh(
        core_axis_name="c", subcore_axis_name="s",
        num_cores=NC, num_subcores=NS,
    )
    k = pl.kernel(
        _gather_body,
        out_type=jax.ShapeDtypeStruct((B, D), jnp.float32),
        mesh=mesh,
        scratch_types=[
            pltpu.VMEM((K, IW), jnp.int32),
            pltpu.VMEM((K * IW, D), jnp.float32),
            pltpu.SemaphoreType.DMA,
        ],
        compiler_params=pltpu.CompilerParams(use_tc_tiling_on_sc=False),
    )
    return k(data, idx2d)


def kernel(data, indices):
    idx2d = indices.astype(jnp.int32).reshape(NROWS, IW)
    out = _gather(data, idx2d)
    return out.reshape(indices.shape[0], indices.shape[1], D)


# speedup vs baseline: 1.1058x; 1.0109x over previous
import jax
import jax.numpy as jnp
from jax import lax
from jax.experimental import pallas as pl
from jax.experimental.pallas import tpu as pltpu
from jax.experimental.pallas import tpu_sc as plsc

NC = 2
NS = 16
NW = NC * NS

D = 32
B = 16384 * 50
IW = 128
NROWS = B // IW
ROWS_PER_W = NROWS // NW
K = 8
NCHUNK = ROWS_PER_W // K
CB = K * IW


def _gather_body(table_hbm, idx_hbm, out_hbm, idx_v, rows_v, gsem, wsem):
    c = lax.axis_index("c")
    s = lax.axis_index("s")
    wid = s * NC + c
    base_row = wid * ROWS_PER_W

    def fire(g, b):
        r0 = base_row + g * K
        pltpu.sync_copy(idx_hbm.at[pl.ds(r0, K)], idx_v.at[b])
        for j in range(K):
            pltpu.async_copy(
                table_hbm.at[idx_v.at[b, j]],
                rows_v.at[b, pl.ds(j * IW, IW)],
                gsem.at[b],
            )

    def wait_gathers(b):
        pltpu.make_async_copy(
            table_hbm.at[pl.ds(0, CB)], rows_v.at[b], gsem.at[b]
        ).wait()

    def wait_writeback(b):
        pltpu.make_async_copy(
            rows_v.at[b], out_hbm.at[pl.ds(0, CB)], wsem.at[b]
        ).wait()

    fire(0, 0)

    def body(g, carry):
        b = lax.rem(g, 2)
        nb = 1 - b

        @pl.when(g + 1 < NCHUNK)
        def _prefetch():
            @pl.when(g >= 1)
            def _drain_prev():
                wait_writeback(nb)

            fire(g + 1, nb)

        wait_gathers(b)
        r0 = base_row + g * K
        pltpu.async_copy(
            rows_v.at[b], out_hbm.at[pl.ds(r0 * IW, CB)], wsem.at[b]
        )
        return carry

    lax.fori_loop(0, NCHUNK, body, 0)
    wait_writeback(0)
    wait_writeback(1)


@jax.jit
def _gather(data, idx2d):
    mesh = plsc.VectorSubcoreMesh(
        core_axis_name="c", subcore_axis_name="s",
        num_cores=NC, num_subcores=NS,
    )
    k = pl.kernel(
        _gather_body,
        out_type=jax.ShapeDtypeStruct((B, D), jnp.float32),
        mesh=mesh,
        scratch_types=[
            pltpu.VMEM((2, K, IW), jnp.int32),
            pltpu.VMEM((2, CB, D), jnp.float32),
            pltpu.SemaphoreType.DMA((2,)),
            pltpu.SemaphoreType.DMA((2,)),
        ],
        compiler_params=pltpu.CompilerParams(use_tc_tiling_on_sc=False),
    )
    return k(data, idx2d)


def kernel(data, indices):
    idx2d = indices.astype(jnp.int32).reshape(NROWS, IW)
    out = _gather(data, idx2d)
    return out.reshape(indices.shape[0], indices.shape[1], D)


# speedup vs baseline: 1.7808x; 1.6104x over previous
import jax
import jax.numpy as jnp
from jax import lax
from jax.experimental import pallas as pl
from jax.experimental.pallas import tpu as pltpu
from jax.experimental.pallas import tpu_sc as plsc

NC = 2
NS = 16
NW = NC * NS

D = 32
NR = 16384
NI = 50
ROWS_PER_W = NR // NW
K = 16
NCHUNK = ROWS_PER_W // K


def _gather_body(table_hbm, idx_hbm, out_hbm, idx_v, rows_v, gsem, wsem):
    c = lax.axis_index("c")
    s = lax.axis_index("s")
    wid = s * NC + c
    base_row = wid * ROWS_PER_W

    def fire(g, b):
        r0 = base_row + g * K
        pltpu.sync_copy(idx_hbm.at[pl.ds(r0, K)], idx_v.at[b])
        for j in range(K):
            pltpu.async_copy(
                table_hbm.at[idx_v.at[b, j]],
                rows_v.at[b, j],
                gsem.at[b],
            )

    def wait_gathers(b):
        pltpu.make_async_copy(
            out_hbm.at[pl.ds(0, K)], rows_v.at[b], gsem.at[b]
        ).wait()

    def wait_writeback(b):
        pltpu.make_async_copy(
            rows_v.at[b], out_hbm.at[pl.ds(0, K)], wsem.at[b]
        ).wait()

    fire(0, 0)

    def body(g, carry):
        b = lax.rem(g, 2)
        nb = 1 - b

        @pl.when(g + 1 < NCHUNK)
        def _prefetch():
            @pl.when(g >= 1)
            def _drain_prev():
                wait_writeback(nb)

            fire(g + 1, nb)

        wait_gathers(b)
        r0 = base_row + g * K
        pltpu.async_copy(
            rows_v.at[b], out_hbm.at[pl.ds(r0, K)], wsem.at[b]
        )
        return carry

    lax.fori_loop(0, NCHUNK, body, 0)
    wait_writeback(0)
    wait_writeback(1)


@jax.jit
def _gather(data, idx):
    mesh = plsc.VectorSubcoreMesh(
        core_axis_name="c", subcore_axis_name="s",
        num_cores=NC, num_subcores=NS,
    )
    k = pl.kernel(
        _gather_body,
        out_type=jax.ShapeDtypeStruct((NR, NI, D), jnp.float32),
        mesh=mesh,
        scratch_types=[
            pltpu.VMEM((2, K, NI), jnp.int32),
            pltpu.VMEM((2, K, NI, D), jnp.float32),
            pltpu.SemaphoreType.DMA((2,)),
            pltpu.SemaphoreType.DMA((2,)),
        ],
        compiler_params=pltpu.CompilerParams(use_tc_tiling_on_sc=False),
    )
    return k(data, idx)


def kernel(data, indices):
    return _gather(data, indices.astype(jnp.int32))
